# Initial kernel scaffold; baseline (speedup 1.0000x reference)
import jax
import jax.numpy as jnp
from jax.experimental import pallas as pl


def kernel(x, table, pe):
    raise NotImplementedError("write your pallas kernel here")


# speedup vs baseline: 4.0210x; 4.0210x over previous
import functools
import math

import jax
import jax.numpy as jnp
from jax import lax
from jax.experimental import pallas as pl
from jax.experimental.pallas import tpu as pltpu
from jax.experimental.pallas import tpu_sc as plsc

_LANES = 16


def _make_sc_kernel(n_tokens, d_model, seq_len, n_workers, chunk, pe_rows):
    tok_per_w = n_tokens // n_workers
    n_chunks = tok_per_w // chunk
    scale = math.sqrt(d_model)
    mesh = plsc.VectorSubcoreMesh(core_axis_name="c", subcore_axis_name="s")

    @functools.partial(
        pl.kernel,
        out_type=jax.ShapeDtypeStruct((n_tokens, d_model), jnp.float32),
        mesh=mesh,
        scratch_types=[
            pltpu.VMEM((n_chunks, chunk), jnp.int32),
            pltpu.VMEM((pe_rows, d_model), jnp.float32),
            pltpu.VMEM((chunk, d_model), jnp.float32),
            pltpu.SemaphoreType.DMA,
        ],
    )
    def k(x_hbm, table_hbm, pe_hbm, out_hbm, idx_v, pe_v, rows_v, sem):
        wid = lax.axis_index("s") * 2 + lax.axis_index("c")
        base = wid * tok_per_w
        pltpu.sync_copy(x_hbm.at[wid], idx_v)
        pltpu.sync_copy(pe_hbm, pe_v)

        def chunk_body(j, carry):
            pltpu.async_copy(table_hbm.at[idx_v.at[j]], rows_v, sem).wait()
            o = lax.rem(j * chunk, seq_len)

            def row_body(r, c2):
                for c in range(d_model // _LANES):
                    sl = pl.ds(c * _LANES, _LANES)
                    rows_v[r, sl] = rows_v[r, sl] * scale + pe_v[o + r, sl]
                return c2

            lax.fori_loop(0, chunk, row_body, 0)
            pltpu.sync_copy(rows_v, out_hbm.at[pl.ds(base + j * chunk, chunk)])
            return carry

        lax.fori_loop(0, n_chunks, chunk_body, 0)

    return k


def kernel(x, table, pe):
    batch, seq_len = x.shape
    vocab, d_model = table.shape
    n_tokens = batch * seq_len
    n_workers = 32
    chunk = 128

    pe_ext = jnp.concatenate([pe[:seq_len], pe[:chunk]], axis=0)
    x3 = x.reshape(n_workers, (n_tokens // n_workers) // chunk, chunk)

    k = _make_sc_kernel(n_tokens, d_model, seq_len, n_workers, chunk,
                        pe_ext.shape[0])
    out = k(x3, table, pe_ext)
    return out.reshape(batch, seq_len, d_model)


# speedup vs baseline: 5.6954x; 1.4164x over previous
import functools
import math

import jax
import jax.numpy as jnp
from jax import lax
from jax.experimental import pallas as pl
from jax.experimental.pallas import tpu as pltpu
from jax.experimental.pallas import tpu_sc as plsc

_LANES = 16
_NBUF = 4


def _make_sc_kernel(n_tokens, d_model, seq_len, n_workers, chunk, pe_rows):
    tok_per_w = n_tokens // n_workers
    n_chunks = tok_per_w // chunk
    n_outer = n_chunks // _NBUF
    scale = math.sqrt(d_model)
    mesh = plsc.VectorSubcoreMesh(core_axis_name="c", subcore_axis_name="s")

    @functools.partial(
        pl.kernel,
        out_type=jax.ShapeDtypeStruct((n_tokens, d_model), jnp.float32),
        mesh=mesh,
        scratch_types=[
            pltpu.VMEM((tok_per_w,), jnp.int32),
            pltpu.VMEM((pe_rows, d_model), jnp.float32),
        ]
        + [pltpu.VMEM((chunk, d_model), jnp.float32) for _ in range(_NBUF)]
        + [pltpu.SemaphoreType.DMA for _ in range(2 * _NBUF)],
    )
    def k(x_hbm, table_hbm, pe_hbm, out_hbm, idx_v, pe_v, *bufs_sems):
        rows = bufs_sems[:_NBUF]
        gsem = bufs_sems[_NBUF:2 * _NBUF]
        ssem = bufs_sems[2 * _NBUF:]
        wid = lax.axis_index("s") * 2 + lax.axis_index("c")
        base = wid * tok_per_w
        pltpu.sync_copy(x_hbm.at[pl.ds(base, tok_per_w)], idx_v)
        pltpu.sync_copy(pe_hbm, pe_v)

        def gather_start(c, b):
            pltpu.async_copy(table_hbm.at[idx_v.at[pl.ds(c * chunk, chunk)]],
                             rows[b], gsem[b])

        def gather_wait(b):
            pltpu.make_async_copy(table_hbm.at[idx_v.at[pl.ds(0, chunk)]],
                                  rows[b], gsem[b]).wait()

        def store_start(c, b):
            pltpu.async_copy(rows[b], out_hbm.at[pl.ds(base + c * chunk, chunk)],
                             ssem[b])

        def store_wait(b):
            pltpu.make_async_copy(rows[b], out_hbm.at[pl.ds(base, chunk)],
                                  ssem[b]).wait()

        def compute(c, b):
            o = lax.rem(c * chunk, seq_len)

            def row_body(r, carry):
                for g in range(d_model // _LANES):
                    sl = pl.ds(g * _LANES, _LANES)
                    rows[b][r, sl] = rows[b][r, sl] * scale + pe_v[o + r, sl]
                return carry

            lax.fori_loop(0, chunk, row_body, 0)

        def step(c, b, first, last):
            gather_wait(b)
            compute(c, b)
            store_start(c, b)
            if not first:
                store_wait((b + _NBUF - 1) % _NBUF)
            if not last:
                gather_start(c + _NBUF - 1, (b + _NBUF - 1) % _NBUF)

        for b in range(_NBUF - 1):
            gather_start(b, b)

        for b in range(_NBUF):
            step(b, b, first=(b == 0), last=False)

        def outer(t, carry):
            for b in range(_NBUF):
                step(t * _NBUF + b, b, first=False, last=False)
            return carry

        lax.fori_loop(1, n_outer - 1, outer, 0)

        for b in range(_NBUF):
            step((n_outer - 1) * _NBUF + b, b, first=False, last=(b > 0))
        store_wait(_NBUF - 1)

    return k


def kernel(x, table, pe):
    batch, seq_len = x.shape
    vocab, d_model = table.shape
    n_tokens = batch * seq_len
    n_workers = 32
    chunk = 80

    pe_ext = jnp.concatenate([pe[:seq_len], pe[:chunk - chunk % 8]], axis=0)
    k = _make_sc_kernel(n_tokens, d_model, seq_len, n_workers, chunk,
                        pe_ext.shape[0])
    out = k(x.reshape(-1), table, pe_ext)
    return out.reshape(batch, seq_len, d_model)


# speedup vs baseline: 5.7542x; 1.0103x over previous
import functools
import math

import jax
import jax.numpy as jnp
from jax import lax
from jax.experimental import pallas as pl
from jax.experimental.pallas import tpu as pltpu
from jax.experimental.pallas import tpu_sc as plsc

_LANES = 16
_NBUF = 8


def _make_sc_kernel(n_tokens, d_model, seq_len, n_workers, chunk, pe_rows):
    tok_per_w = n_tokens // n_workers
    n_chunks = tok_per_w // chunk
    n_outer = n_chunks // _NBUF
    scale = math.sqrt(d_model)
    mesh = plsc.VectorSubcoreMesh(core_axis_name="c", subcore_axis_name="s")

    @functools.partial(
        pl.kernel,
        out_type=jax.ShapeDtypeStruct((n_tokens, d_model), jnp.float32),
        mesh=mesh,
        scratch_types=[
            pltpu.VMEM((tok_per_w,), jnp.int32),
            pltpu.VMEM((pe_rows, d_model), jnp.float32),
        ]
        + [pltpu.VMEM((chunk, d_model), jnp.float32) for _ in range(_NBUF)]
        + [pltpu.SemaphoreType.DMA for _ in range(2 * _NBUF)],
    )
    def k(x_hbm, table_hbm, pe_hbm, out_hbm, idx_v, pe_v, *bufs_sems):
        rows = bufs_sems[:_NBUF]
        gsem = bufs_sems[_NBUF:2 * _NBUF]
        ssem = bufs_sems[2 * _NBUF:]
        wid = lax.axis_index("s") * 2 + lax.axis_index("c")
        base = wid * tok_per_w
        pltpu.sync_copy(x_hbm.at[pl.ds(base, tok_per_w)], idx_v)
        pltpu.sync_copy(pe_hbm, pe_v)

        def gather_start(c, b):
            pltpu.async_copy(table_hbm.at[idx_v.at[pl.ds(c * chunk, chunk)]],
                             rows[b], gsem[b])

        def gather_wait(b):
            pltpu.make_async_copy(table_hbm.at[idx_v.at[pl.ds(0, chunk)]],
                                  rows[b], gsem[b]).wait()

        def store_start(c, b):
            pltpu.async_copy(rows[b], out_hbm.at[pl.ds(base + c * chunk, chunk)],
                             ssem[b])

        def store_wait(b):
            pltpu.make_async_copy(rows[b], out_hbm.at[pl.ds(base, chunk)],
                                  ssem[b]).wait()

        def compute(c, b):
            o = lax.rem(c * chunk, seq_len)

            def row_body(r, carry):
                for g in range(d_model // _LANES):
                    sl = pl.ds(g * _LANES, _LANES)
                    rows[b][r, sl] = rows[b][r, sl] * scale + pe_v[o + r, sl]
                return carry

            lax.fori_loop(0, chunk, row_body, 0)

        def step(c, b, first, last):
            gather_wait(b)
            compute(c, b)
            store_start(c, b)
            if not first:
                store_wait((b + _NBUF - 1) % _NBUF)
            if not last:
                gather_start(c + _NBUF - 1, (b + _NBUF - 1) % _NBUF)

        for b in range(_NBUF - 1):
            gather_start(b, b)

        for b in range(_NBUF):
            step(b, b, first=(b == 0), last=False)

        def outer(t, carry):
            for b in range(_NBUF):
                step(t * _NBUF + b, b, first=False, last=False)
            return carry

        lax.fori_loop(1, n_outer - 1, outer, 0)

        for b in range(_NBUF):
            step((n_outer - 1) * _NBUF + b, b, first=False, last=(b > 0))
        store_wait(_NBUF - 1)

    return k


def kernel(x, table, pe):
    batch, seq_len = x.shape
    vocab, d_model = table.shape
    n_tokens = batch * seq_len
    n_workers = 32
    chunk = 64

    wrap = chunk - math.gcd(chunk, seq_len)
    pe_ext = jnp.concatenate([pe[:seq_len], pe[:wrap]], axis=0)
    k = _make_sc_kernel(n_tokens, d_model, seq_len, n_workers, chunk,
                        pe_ext.shape[0])
    out = k(x.reshape(-1), table, pe_ext)
    return out.reshape(batch, seq_len, d_model)


# speedup vs baseline: 6.6199x; 1.1505x over previous
import functools
import math

import jax
import jax.numpy as jnp
from jax import lax
from jax.experimental import pallas as pl
from jax.experimental.pallas import tpu as pltpu
from jax.experimental.pallas import tpu_sc as plsc

_LANES = 16
_NBUF = 8


def _make_sc_kernel(n_tokens, d_model, seq_len, n_workers, chunk, pe_rows):
    tok_per_w = n_tokens // n_workers
    n_chunks = tok_per_w // chunk
    n_outer = n_chunks // _NBUF
    scale = math.sqrt(d_model)
    mesh = plsc.VectorSubcoreMesh(core_axis_name="c", subcore_axis_name="s")

    @functools.partial(
        pl.kernel,
        out_type=jax.ShapeDtypeStruct((n_tokens, d_model), jnp.float32),
        mesh=mesh,
        scratch_types=[
            pltpu.VMEM((tok_per_w,), jnp.int32),
            pltpu.VMEM((pe_rows, d_model), jnp.float32),
        ]
        + [pltpu.VMEM((chunk, d_model), jnp.float32) for _ in range(_NBUF)]
        + [pltpu.SemaphoreType.DMA for _ in range(2 * _NBUF)],
    )
    def k(x_hbm, table_hbm, pe_hbm, out_hbm, idx_v, pe_v, *bufs_sems):
        rows = bufs_sems[:_NBUF]
        gsem = bufs_sems[_NBUF:2 * _NBUF]
        ssem = bufs_sems[2 * _NBUF:]
        wid = lax.axis_index("s") * 2 + lax.axis_index("c")
        base = wid * tok_per_w
        pltpu.sync_copy(x_hbm.at[pl.ds(base, tok_per_w)], idx_v)
        pltpu.sync_copy(pe_hbm, pe_v)

        def gather_start(c, b):
            pltpu.async_copy(table_hbm.at[idx_v.at[pl.ds(c * chunk, chunk)]],
                             rows[b], gsem[b])

        def gather_wait(b):
            pltpu.make_async_copy(table_hbm.at[idx_v.at[pl.ds(0, chunk)]],
                                  rows[b], gsem[b]).wait()

        def store_start(c, b):
            pltpu.async_copy(rows[b], out_hbm.at[pl.ds(base + c * chunk, chunk)],
                             ssem[b])

        def store_wait(b):
            pltpu.make_async_copy(rows[b], out_hbm.at[pl.ds(base, chunk)],
                                  ssem[b]).wait()

        def compute(c, b):
            o = lax.rem(c * chunk, seq_len)

            def row_body(r, carry):
                for g in range(d_model // _LANES):
                    sl = pl.ds(g * _LANES, _LANES)
                    plsc.addupdate(rows[b].at[r, sl], pe_v[o + r, sl])
                return carry

            lax.fori_loop(0, chunk, row_body, 0)

        def step(c, b, first, last):
            gather_wait(b)
            compute(c, b)
            store_start(c, b)
            if not first:
                store_wait((b + _NBUF - 1) % _NBUF)
            if not last:
                gather_start(c + _NBUF - 1, (b + _NBUF - 1) % _NBUF)

        for b in range(_NBUF - 1):
            gather_start(b, b)

        for b in range(_NBUF):
            step(b, b, first=(b == 0), last=False)

        def outer(t, carry):
            for b in range(_NBUF):
                step(t * _NBUF + b, b, first=False, last=False)
            return carry

        lax.fori_loop(1, n_outer - 1, outer, 0)

        for b in range(_NBUF):
            step((n_outer - 1) * _NBUF + b, b, first=False, last=(b > 0))
        store_wait(_NBUF - 1)

    return k


def _prescale_table(table, scale):
    vocab, d_model = table.shape
    rows_per_block = 1000
    assert vocab % rows_per_block == 0

    def body(t_ref, o_ref):
        o_ref[...] = t_ref[...] * scale

    return pl.pallas_call(
        body,
        grid=(vocab // rows_per_block,),
        in_specs=[pl.BlockSpec((rows_per_block, d_model), lambda i: (i, 0))],
        out_specs=pl.BlockSpec((rows_per_block, d_model), lambda i: (i, 0)),
        out_shape=jax.ShapeDtypeStruct((vocab, d_model), jnp.float32),
    )(table)


def kernel(x, table, pe):
    batch, seq_len = x.shape
    vocab, d_model = table.shape
    n_tokens = batch * seq_len
    n_workers = 32
    chunk = 64

    table_s = _prescale_table(table, math.sqrt(d_model))
    wrap = chunk - math.gcd(chunk, seq_len)
    pe_ext = jnp.concatenate([pe[:seq_len], pe[:wrap]], axis=0)
    k = _make_sc_kernel(n_tokens, d_model, seq_len, n_workers, chunk,
                        pe_ext.shape[0])
    out = k(x.reshape(-1), table_s, pe_ext)
    return out.reshape(batch, seq_len, d_model)


# speedup vs baseline: 7.9869x; 1.2065x over previous
import functools
import math

import jax
import jax.numpy as jnp
from jax import lax
from jax.experimental import pallas as pl
from jax.experimental.pallas import tpu as pltpu
from jax.experimental.pallas import tpu_sc as plsc

_NBUF = 8


def _make_sc_gather(n_tokens, d_model, n_workers, chunk):
    tok_per_w = n_tokens // n_workers
    n_chunks = tok_per_w // chunk
    n_outer = n_chunks // _NBUF
    mesh = plsc.VectorSubcoreMesh(core_axis_name="c", subcore_axis_name="s")

    @functools.partial(
        pl.kernel,
        out_type=jax.ShapeDtypeStruct((n_tokens, d_model), jnp.float32),
        mesh=mesh,
        scratch_types=[
            pltpu.VMEM((tok_per_w,), jnp.int32),
        ]
        + [pltpu.VMEM((chunk, d_model), jnp.float32) for _ in range(_NBUF)]
        + [pltpu.SemaphoreType.DMA for _ in range(2 * _NBUF)],
    )
    def k(x_hbm, table_hbm, out_hbm, idx_v, *bufs_sems):
        rows = bufs_sems[:_NBUF]
        gsem = bufs_sems[_NBUF:2 * _NBUF]
        ssem = bufs_sems[2 * _NBUF:]
        wid = lax.axis_index("s") * 2 + lax.axis_index("c")
        base = wid * tok_per_w
        pltpu.sync_copy(x_hbm.at[pl.ds(base, tok_per_w)], idx_v)

        def gather_start(c, b):
            pltpu.async_copy(table_hbm.at[idx_v.at[pl.ds(c * chunk, chunk)]],
                             rows[b], gsem[b])

        def gather_wait(b):
            pltpu.make_async_copy(table_hbm.at[idx_v.at[pl.ds(0, chunk)]],
                                  rows[b], gsem[b]).wait()

        def store_start(c, b):
            pltpu.async_copy(rows[b], out_hbm.at[pl.ds(base + c * chunk, chunk)],
                             ssem[b])

        def store_wait(b):
            pltpu.make_async_copy(rows[b], out_hbm.at[pl.ds(base, chunk)],
                                  ssem[b]).wait()

        def step(c, b, first, last):
            gather_wait(b)
            store_start(c, b)
            if not first:
                store_wait((b + _NBUF - 1) % _NBUF)
            if not last:
                gather_start(c + _NBUF - 1, (b + _NBUF - 1) % _NBUF)

        for b in range(_NBUF - 1):
            gather_start(b, b)

        for b in range(_NBUF):
            step(b, b, first=(b == 0), last=False)

        def outer(t, carry):
            for b in range(_NBUF):
                step(t * _NBUF + b, b, False, False)
            return carry

        lax.fori_loop(1, n_outer - 1, outer, 0)

        for b in range(_NBUF):
            step((n_outer - 1) * _NBUF + b, b, first=False, last=(b > 0))
        store_wait(_NBUF - 1)

    return k


def _scale_add_pe(raw, pe_tile, scale, seqs_per_block):
    n_tokens, d_model = raw.shape
    block = pe_tile.shape[0]
    assert n_tokens % block == 0

    def body(r_ref, pe_ref, o_ref):
        o_ref[...] = r_ref[...] * scale + pe_ref[...]

    return pl.pallas_call(
        body,
        grid=(n_tokens // block,),
        in_specs=[
            pl.BlockSpec((block, d_model), lambda i: (i, 0)),
            pl.BlockSpec((block, d_model), lambda i: (0, 0)),
        ],
        out_specs=pl.BlockSpec((block, d_model), lambda i: (i, 0)),
        out_shape=jax.ShapeDtypeStruct((n_tokens, d_model), jnp.float32),
    )(raw, pe_tile)


def kernel(x, table, pe):
    batch, seq_len = x.shape
    vocab, d_model = table.shape
    n_tokens = batch * seq_len
    n_workers = 32
    chunk = 64
    seqs_per_block = 8

    gather = _make_sc_gather(n_tokens, d_model, n_workers, chunk)
    raw = gather(x.reshape(-1), table)
    pe_tile = jnp.tile(pe[:seq_len], (seqs_per_block, 1))
    out = _scale_add_pe(raw, pe_tile, math.sqrt(d_model), seqs_per_block)
    return out.reshape(batch, seq_len, d_model)


# speedup vs baseline: 8.7503x; 1.0956x over previous
import functools
import math

import jax
import jax.numpy as jnp
from jax import lax
from jax.experimental import pallas as pl
from jax.experimental.pallas import tpu as pltpu
from jax.experimental.pallas import tpu_sc as plsc

_NBUF = 4
_STAGES = 4


def _make_sc_gather(n_tokens, d_model, n_workers, chunk):
    tok_per_w = n_tokens // n_workers
    n_chunks = tok_per_w // chunk
    n_outer = n_chunks // _NBUF
    mesh = plsc.VectorSubcoreMesh(core_axis_name="c", subcore_axis_name="s")

    @functools.partial(
        pl.kernel,
        out_type=jax.ShapeDtypeStruct((n_tokens, d_model), jnp.float32),
        mesh=mesh,
        scratch_types=[
            pltpu.VMEM((tok_per_w,), jnp.int32),
        ]
        + [pltpu.VMEM((chunk, d_model), jnp.float32) for _ in range(_NBUF)]
        + [pltpu.SemaphoreType.DMA for _ in range(2 * _NBUF)],
    )
    def k(x_hbm, table_hbm, out_hbm, idx_v, *bufs_sems):
        rows = bufs_sems[:_NBUF]
        gsem = bufs_sems[_NBUF:2 * _NBUF]
        ssem = bufs_sems[2 * _NBUF:]
        wid = lax.axis_index("s") * 2 + lax.axis_index("c")
        base = wid * tok_per_w
        pltpu.sync_copy(x_hbm.at[pl.ds(base, tok_per_w)], idx_v)

        def gather_start(c, b):
            pltpu.async_copy(table_hbm.at[idx_v.at[pl.ds(c * chunk, chunk)]],
                             rows[b], gsem[b])

        def gather_wait(b):
            pltpu.make_async_copy(table_hbm.at[idx_v.at[pl.ds(0, chunk)]],
                                  rows[b], gsem[b]).wait()

        def store_start(c, b):
            pltpu.async_copy(rows[b], out_hbm.at[pl.ds(base + c * chunk, chunk)],
                             ssem[b])

        def store_wait(b):
            pltpu.make_async_copy(rows[b], out_hbm.at[pl.ds(base, chunk)],
                                  ssem[b]).wait()

        def step(c, b, first, last):
            gather_wait(b)
            store_start(c, b)
            if not first:
                store_wait((b + _NBUF - 1) % _NBUF)
            if not last:
                gather_start(c + _NBUF - 1, (b + _NBUF - 1) % _NBUF)

        for b in range(_NBUF - 1):
            gather_start(b, b)

        for b in range(_NBUF):
            step(b, b, first=(b == 0), last=False)

        def outer(t, carry):
            for b in range(_NBUF):
                step(t * _NBUF + b, b, False, False)
            return carry

        lax.fori_loop(1, n_outer - 1, outer, 0)

        for b in range(_NBUF):
            step((n_outer - 1) * _NBUF + b, b, first=False, last=(b > 0))
        store_wait(_NBUF - 1)

    return k


def _scale_add_pe_stage(raw_part, pe_tile, prev_out, n_tokens, block_base,
                        scale):
    n_part, d_model = raw_part.shape
    block = pe_tile.shape[0]
    assert n_part % block == 0

    def body(r_ref, pe_ref, *rest):
        o_ref = rest[-1]
        o_ref[...] = r_ref[...] * scale + pe_ref[...]

    in_specs = [
        pl.BlockSpec((block, d_model), lambda i: (i, 0)),
        pl.BlockSpec((block, d_model), lambda i: (0, 0)),
    ]
    args = [raw_part, pe_tile]
    aliases = {}
    if prev_out is not None:
        in_specs.append(pl.BlockSpec(memory_space=pl.ANY))
        args.append(prev_out)
        aliases = {2: 0}

    return pl.pallas_call(
        body,
        grid=(n_part // block,),
        in_specs=in_specs,
        out_specs=pl.BlockSpec((block, d_model),
                               lambda i: (block_base + i, 0)),
        out_shape=jax.ShapeDtypeStruct((n_tokens, d_model), jnp.float32),
        input_output_aliases=aliases,
    )(*args)


def kernel(x, table, pe):
    batch, seq_len = x.shape
    vocab, d_model = table.shape
    n_tokens = batch * seq_len
    n_workers = 32
    chunk = 64
    seqs_per_block = 8
    block = seqs_per_block * seq_len
    part = n_tokens // _STAGES
    scale = math.sqrt(d_model)

    gather = _make_sc_gather(part, d_model, n_workers, chunk)
    x_flat = x.reshape(-1)
    raws = [gather(lax.slice(x_flat, (i * part,), ((i + 1) * part,)), table)
            for i in range(_STAGES)]
    pe_tile = jnp.tile(pe[:seq_len], (seqs_per_block, 1))
    out = None
    for i in range(_STAGES):
        out = _scale_add_pe_stage(raws[i], pe_tile, out, n_tokens,
                                  i * (part // block), scale)
    return out.reshape(batch, seq_len, d_model)


# speedup vs baseline: 8.8197x; 1.0079x over previous
import functools
import math

import jax
import jax.numpy as jnp
from jax import lax
from jax.experimental import pallas as pl
from jax.experimental.pallas import tpu as pltpu
from jax.experimental.pallas import tpu_sc as plsc

_NBUF = 5
_STAGES = 8


def _make_sc_gather(n_tokens, d_model, n_workers, chunk):
    tok_per_w = n_tokens // n_workers
    n_chunks = tok_per_w // chunk
    n_outer = n_chunks // _NBUF
    mesh = plsc.VectorSubcoreMesh(core_axis_name="c", subcore_axis_name="s")

    @functools.partial(
        pl.kernel,
        out_type=jax.ShapeDtypeStruct((n_tokens, d_model), jnp.float32),
        mesh=mesh,
        scratch_types=[
            pltpu.VMEM((tok_per_w,), jnp.int32),
        ]
        + [pltpu.VMEM((chunk, d_model), jnp.float32) for _ in range(_NBUF)]
        + [pltpu.SemaphoreType.DMA for _ in range(2 * _NBUF)],
    )
    def k(x_hbm, table_hbm, out_hbm, idx_v, *bufs_sems):
        rows = bufs_sems[:_NBUF]
        gsem = bufs_sems[_NBUF:2 * _NBUF]
        ssem = bufs_sems[2 * _NBUF:]
        wid = lax.axis_index("s") * 2 + lax.axis_index("c")
        base = wid * tok_per_w
        pltpu.sync_copy(x_hbm.at[pl.ds(base, tok_per_w)], idx_v)

        def gather_start(c, b):
            pltpu.async_copy(table_hbm.at[idx_v.at[pl.ds(c * chunk, chunk)]],
                             rows[b], gsem[b])

        def gather_wait(b):
            pltpu.make_async_copy(table_hbm.at[idx_v.at[pl.ds(0, chunk)]],
                                  rows[b], gsem[b]).wait()

        def store_start(c, b):
            pltpu.async_copy(rows[b], out_hbm.at[pl.ds(base + c * chunk, chunk)],
                             ssem[b])

        def store_wait(b):
            pltpu.make_async_copy(rows[b], out_hbm.at[pl.ds(base, chunk)],
                                  ssem[b]).wait()

        def step(c, b, first, last):
            gather_wait(b)
            store_start(c, b)
            if not first:
                store_wait((b + _NBUF - 1) % _NBUF)
            if not last:
                gather_start(c + _NBUF - 1, (b + _NBUF - 1) % _NBUF)

        for b in range(_NBUF - 1):
            gather_start(b, b)

        for b in range(_NBUF):
            step(b, b, first=(b == 0), last=False)

        def outer(t, carry):
            for b in range(_NBUF):
                step(t * _NBUF + b, b, False, False)
            return carry

        lax.fori_loop(1, n_outer - 1, outer, 0)

        for b in range(_NBUF):
            step((n_outer - 1) * _NBUF + b, b, first=False, last=(b > 0))
        store_wait(_NBUF - 1)

    return k


def _scale_add_pe_stage(raw_part, pe_tile, prev_out, n_tokens, block_base,
                        scale):
    n_part, d_model = raw_part.shape
    block = pe_tile.shape[0]
    assert n_part % block == 0

    def body(r_ref, pe_ref, *rest):
        o_ref = rest[-1]
        o_ref[...] = r_ref[...] * scale + pe_ref[...]

    in_specs = [
        pl.BlockSpec((block, d_model), lambda i: (i, 0)),
        pl.BlockSpec((block, d_model), lambda i: (0, 0)),
    ]
    args = [raw_part, pe_tile]
    aliases = {}
    if prev_out is not None:
        in_specs.append(pl.BlockSpec(memory_space=pl.ANY))
        args.append(prev_out)
        aliases = {2: 0}

    return pl.pallas_call(
        body,
        grid=(n_part // block,),
        in_specs=in_specs,
        out_specs=pl.BlockSpec((block, d_model),
                               lambda i: (block_base + i, 0)),
        out_shape=jax.ShapeDtypeStruct((n_tokens, d_model), jnp.float32),
        input_output_aliases=aliases,
    )(*args)


def kernel(x, table, pe):
    batch, seq_len = x.shape
    vocab, d_model = table.shape
    n_tokens = batch * seq_len
    n_workers = 32
    chunk = 64
    seqs_per_block = 8
    block = seqs_per_block * seq_len
    part = n_tokens // _STAGES
    scale = math.sqrt(d_model)

    gather = _make_sc_gather(part, d_model, n_workers, chunk)
    x_flat = x.reshape(-1)
    raws = [gather(lax.slice(x_flat, (i * part,), ((i + 1) * part,)), table)
            for i in range(_STAGES)]
    pe_tile = jnp.tile(pe[:seq_len], (seqs_per_block, 1))
    out = None
    for i in range(_STAGES):
        out = _scale_add_pe_stage(raws[i], pe_tile, out, n_tokens,
                                  i * (part // block), scale)
    return out.reshape(batch, seq_len, d_model)
